# baseline (speedup 1.0000x reference)
import jax
import jax.numpy as jnp
from jax.experimental import pallas as pl
from jax.experimental.pallas import tpu as pltpu

_DYN = 96
_STAT = 32
_U = 8
_NOBS = 13
_HZ = 128
_H1 = 200
_H2 = 200
_LANE = 128

_TB = 512


def _seg_sum_matrix(n_cols, period, n_out):
    return (jnp.arange(n_cols)[:, None] // period
            == jnp.arange(n_out)[None, :]).astype(jnp.bfloat16)


def _fused_body(zd_ref, zs_ref, dt_ref, ut_ref,
                w1d_ref, w1s_ref, w1t_ref, b1_ref,
                w2_ref, b2_ref, w3_ref, b3_ref,
                wa_ref, ba_ref, wb_ref, bb_ref,
                wc_ref, bc_ref, wd_ref, bd_ref,
                sa_ref, sb_ref, sc_ref, sd_ref,
                z_ref, y_ref,
                w1d_bf, w1s_bf, w2_bf, w3_bf,
                wa_bf, wb_bf, wc_bf, wd_bf):
    bf16 = jnp.bfloat16
    f32 = jnp.float32

    @pl.when(pl.program_id(0) == 0)
    def _cast_weights():
        w1d_bf[...] = w1d_ref[...].astype(bf16)
        w1s_bf[...] = w1s_ref[...].astype(bf16)
        w2_bf[...] = w2_ref[...].astype(bf16)
        w3_bf[...] = w3_ref[...].astype(bf16)
        wa_bf[...] = wa_ref[...].astype(bf16)
        wb_bf[...] = wb_ref[...].astype(bf16)
        wc_bf[...] = wc_ref[...].astype(bf16)
        wd_bf[...] = wd_ref[...].astype(bf16)

    zd = zd_ref[...]
    dt = dt_ref[...]
    h = (jnp.dot(zd.astype(bf16), w1d_bf[...], preferred_element_type=f32)
         + jnp.dot(zs_ref[...].astype(bf16), w1s_bf[...],
                   preferred_element_type=f32)
         + dt * w1t_ref[...] + b1_ref[...])
    h = jnp.maximum(h, 0.0)
    h = jnp.maximum(
        jnp.dot(h.astype(bf16), w2_bf[...], preferred_element_type=f32)
        + b2_ref[...], 0.0)
    hz = (jnp.dot(h.astype(bf16), w3_bf[...], preferred_element_type=f32)
          + b3_ref[...])
    hzb = hz.astype(bf16)

    utdt = ut_ref[...] * dt
    u128 = jnp.concatenate([utdt] * 16, axis=1)

    ab = (jnp.dot(hzb, wa_bf[...], preferred_element_type=f32)
          + ba_ref[...])
    z384 = jnp.concatenate([zd] * 4, axis=1)
    p_a = jnp.concatenate(
        [ab[:, k * 384:(k + 1) * 384] * z384 for k in range(24)],
        axis=1).astype(bf16)
    z_part = jnp.dot(p_a, sa_ref[...], preferred_element_type=f32)

    bt = (jnp.dot(hzb, wb_bf[...], preferred_element_type=f32)
          + bb_ref[...])
    p_b = jnp.concatenate(
        [bt[:, k * 128:(k + 1) * 128] * u128 for k in range(6)],
        axis=1).astype(bf16)
    zfull = z_part + jnp.dot(p_b, sb_ref[...],
                             preferred_element_type=f32)

    zn384 = jnp.concatenate([zfull[:, :_DYN]] * 4, axis=1)

    ct = (jnp.dot(hzb, wc_bf[...], preferred_element_type=f32)
          + bc_ref[...])
    p_c = jnp.concatenate(
        [ct[:, k * 384:(k + 1) * 384] * zn384 for k in range(3)]
        + [ct[:, 1152:1248] * zfull[:, :_DYN]],
        axis=1).astype(bf16)
    yt = jnp.dot(p_c, sc_ref[...], preferred_element_type=f32)

    dtv = (jnp.dot(hzb, wd_bf[...], preferred_element_type=f32)
           + bd_ref[...])
    p_d = (dtv * u128[:, :_NOBS * _U]).astype(bf16)
    yt = yt + jnp.dot(p_d, sd_ref[...], preferred_element_type=f32)

    z_ref[...] = zfull[:, :_DYN]
    y_ref[...] = yt


@jax.jit
def kernel(z_dyn, z_static, dt, ut, W1, b1, W2, b2, W3, b3,
           Wa, ba, Wb, bb, Wc, bc, Wd, bd):
    B = z_dyn.shape[0]
    f32 = jnp.float32
    bf16 = jnp.bfloat16

    W1d = W1[:_DYN]
    W1s = W1[_DYN:_DYN + _STAT]
    W1t = W1[_DYN + _STAT:]

    sa = _seg_sum_matrix(_DYN * _DYN, _DYN, _LANE)
    sb = _seg_sum_matrix(_DYN * _U, _U, _LANE)
    sc = _seg_sum_matrix(_NOBS * _DYN, _DYN, _NOBS)
    sd = _seg_sum_matrix(_NOBS * _U, _U, _NOBS)

    grid = (B // _TB,)
    row_spec = lambda n: pl.BlockSpec((_TB, n), lambda i: (i, 0))
    w_spec = lambda shp: pl.BlockSpec(shp, lambda i: (0,) * len(shp))

    z_next, yt = pl.pallas_call(
        _fused_body,
        grid=grid,
        in_specs=[
            row_spec(_DYN),
            row_spec(_STAT),
            row_spec(1),
            row_spec(_U),
            w_spec((_DYN, _H1)), w_spec((_STAT, _H1)), w_spec((1, _H1)),
            w_spec((1, _H1)),
            w_spec((_H1, _H2)), w_spec((1, _H2)),
            w_spec((_H2, _HZ)), w_spec((1, _HZ)),
            w_spec((_HZ, _DYN * _DYN)), w_spec((1, _DYN * _DYN)),
            w_spec((_HZ, _DYN * _U)), w_spec((1, _DYN * _U)),
            w_spec((_HZ, _NOBS * _DYN)), w_spec((1, _NOBS * _DYN)),
            w_spec((_HZ, _NOBS * _U)), w_spec((1, _NOBS * _U)),
            w_spec((_DYN * _DYN, _LANE)),
            w_spec((_DYN * _U, _LANE)),
            w_spec((_NOBS * _DYN, _NOBS)),
            w_spec((_NOBS * _U, _NOBS)),
        ],
        out_specs=[row_spec(_DYN), row_spec(_NOBS)],
        out_shape=[
            jax.ShapeDtypeStruct((B, _DYN), f32),
            jax.ShapeDtypeStruct((B, _NOBS), f32),
        ],
        scratch_shapes=[
            pltpu.VMEM((_DYN, _H1), bf16),
            pltpu.VMEM((_STAT, _H1), bf16),
            pltpu.VMEM((_H1, _H2), bf16),
            pltpu.VMEM((_H2, _HZ), bf16),
            pltpu.VMEM((_HZ, _DYN * _DYN), bf16),
            pltpu.VMEM((_HZ, _DYN * _U), bf16),
            pltpu.VMEM((_HZ, _NOBS * _DYN), bf16),
            pltpu.VMEM((_HZ, _NOBS * _U), bf16),
        ],
        compiler_params=pltpu.CompilerParams(
            dimension_semantics=("arbitrary",)),
    )(z_dyn, z_static, dt, ut,
      W1d, W1s, W1t, b1.reshape(1, -1),
      W2, b2.reshape(1, -1),
      W3, b3.reshape(1, -1),
      Wa, ba.reshape(1, -1),
      Wb, bb.reshape(1, -1),
      Wc, bc.reshape(1, -1),
      Wd, bd.reshape(1, -1),
      sa, sb, sc, sd)

    return (z_next, yt)


# speedup vs baseline: 1.3234x; 1.3234x over previous
import jax
import jax.numpy as jnp
from jax.experimental import pallas as pl
from jax.experimental.pallas import tpu as pltpu

_DYN = 96
_STAT = 32
_U = 8
_NOBS = 13
_HZ = 128
_H1 = 200
_H2 = 200
_LANE = 128

_TB = 256


def _seg_sum_matrix(n_cols, period, n_out):
    return (jnp.arange(n_cols)[:, None] // period
            == jnp.arange(n_out)[None, :]).astype(jnp.bfloat16)


def _fused_body(zd_ref, zs_ref, dt_ref, ut_ref,
                w1d_ref, w1s_ref, w1t_ref, b1_ref,
                w2_ref, b2_ref, w3_ref, b3_ref,
                wa_ref, wb_ref, wc_ref, wd_ref,
                baT_ref, bbT_ref, bcT_ref, bdT_ref,
                sa_ref, sb_ref, sc_ref, sd_ref,
                z_ref, y_ref,
                w1d_bf, w1s_bf, w2_bf, w3_bf,
                wa_bf, wb_bf, wc_bf, wd_bf):
    bf16 = jnp.bfloat16
    f32 = jnp.float32

    @pl.when(pl.program_id(0) == 0)
    def _cast_weights():
        w1d_bf[...] = w1d_ref[...].astype(bf16)
        w1s_bf[...] = w1s_ref[...].astype(bf16)
        w2_bf[...] = w2_ref[...].astype(bf16)
        w3_bf[...] = w3_ref[...].astype(bf16)
        wa_bf[...] = wa_ref[...].astype(bf16)
        wb_bf[...] = wb_ref[...].astype(bf16)
        wc_bf[...] = wc_ref[...].astype(bf16)
        wd_bf[...] = wd_ref[...].astype(bf16)

    zd = zd_ref[...]
    dt = dt_ref[...]
    h = (jnp.dot(zd.astype(bf16), w1d_bf[...], preferred_element_type=f32)
         + jnp.dot(zs_ref[...].astype(bf16), w1s_bf[...],
                   preferred_element_type=f32)
         + dt * w1t_ref[...] + b1_ref[...])
    h = jnp.maximum(h, 0.0)
    h = jnp.maximum(
        jnp.dot(h.astype(bf16), w2_bf[...], preferred_element_type=f32)
        + b2_ref[...], 0.0)
    hz = (jnp.dot(h.astype(bf16), w3_bf[...], preferred_element_type=f32)
          + b3_ref[...])
    hzb = hz.astype(bf16)

    utdt = ut_ref[...] * dt
    u128 = jnp.concatenate([utdt] * 16, axis=1).astype(bf16)

    ab = jnp.dot(hzb, wa_bf[...],
                 preferred_element_type=f32).astype(bf16)
    z384 = jnp.concatenate([zd] * 4, axis=1).astype(bf16)
    p_a = jnp.concatenate(
        [ab[:, k * 384:(k + 1) * 384] * z384 for k in range(24)], axis=1)
    z_part = jnp.dot(p_a, sa_ref[...], preferred_element_type=f32)

    bt = jnp.dot(hzb, wb_bf[...],
                 preferred_element_type=f32).astype(bf16)
    p_b = jnp.concatenate(
        [bt[:, k * 128:(k + 1) * 128] * u128 for k in range(6)], axis=1)
    z_part = z_part + jnp.dot(p_b, sb_ref[...], preferred_element_type=f32)

    zfull = (z_part
             + jnp.dot(zd, baT_ref[...], preferred_element_type=f32)
             + jnp.dot(utdt, bbT_ref[...], preferred_element_type=f32))

    zn = zfull[:, :_DYN]
    zn384 = jnp.concatenate([zn] * 4, axis=1).astype(bf16)

    ct = jnp.dot(hzb, wc_bf[...],
                 preferred_element_type=f32).astype(bf16)
    p_c = jnp.concatenate(
        [ct[:, k * 384:(k + 1) * 384] * zn384 for k in range(3)]
        + [ct[:, 1152:1248] * zn384[:, :_DYN]], axis=1)
    yt = jnp.dot(p_c, sc_ref[...], preferred_element_type=f32)

    dtv = jnp.dot(hzb, wd_bf[...],
                  preferred_element_type=f32).astype(bf16)
    p_d = dtv * u128[:, :_NOBS * _U]
    yt = (yt + jnp.dot(p_d, sd_ref[...], preferred_element_type=f32)
          + jnp.dot(zn, bcT_ref[...], preferred_element_type=f32)
          + jnp.dot(utdt, bdT_ref[...], preferred_element_type=f32))

    z_ref[...] = zfull[:, :_DYN]
    y_ref[...] = yt


@jax.jit
def kernel(z_dyn, z_static, dt, ut, W1, b1, W2, b2, W3, b3,
           Wa, ba, Wb, bb, Wc, bc, Wd, bd):
    B = z_dyn.shape[0]
    f32 = jnp.float32
    bf16 = jnp.bfloat16

    W1d = W1[:_DYN]
    W1s = W1[_DYN:_DYN + _STAT]
    W1t = W1[_DYN + _STAT:]

    baT = jnp.pad(ba.reshape(_DYN, _DYN).T, ((0, 0), (0, _LANE - _DYN)))
    bbT = jnp.pad(bb.reshape(_DYN, _U).T, ((0, 0), (0, _LANE - _DYN)))
    bcT = bc.reshape(_NOBS, _DYN).T
    bdT = bd.reshape(_NOBS, _U).T

    sa = _seg_sum_matrix(_DYN * _DYN, _DYN, _LANE)
    sb = _seg_sum_matrix(_DYN * _U, _U, _LANE)
    sc = _seg_sum_matrix(_NOBS * _DYN, _DYN, _NOBS)
    sd = _seg_sum_matrix(_NOBS * _U, _U, _NOBS)

    grid = (B // _TB,)
    row_spec = lambda n: pl.BlockSpec((_TB, n), lambda i: (i, 0))
    w_spec = lambda shp: pl.BlockSpec(shp, lambda i: (0,) * len(shp))

    z_next, yt = pl.pallas_call(
        _fused_body,
        grid=grid,
        in_specs=[
            row_spec(_DYN),
            row_spec(_STAT),
            row_spec(1),
            row_spec(_U),
            w_spec((_DYN, _H1)), w_spec((_STAT, _H1)), w_spec((1, _H1)),
            w_spec((1, _H1)),
            w_spec((_H1, _H2)), w_spec((1, _H2)),
            w_spec((_H2, _HZ)), w_spec((1, _HZ)),
            w_spec((_HZ, _DYN * _DYN)),
            w_spec((_HZ, _DYN * _U)),
            w_spec((_HZ, _NOBS * _DYN)),
            w_spec((_HZ, _NOBS * _U)),
            w_spec((_DYN, _LANE)), w_spec((_U, _LANE)),
            w_spec((_DYN, _NOBS)), w_spec((_U, _NOBS)),
            w_spec((_DYN * _DYN, _LANE)),
            w_spec((_DYN * _U, _LANE)),
            w_spec((_NOBS * _DYN, _NOBS)),
            w_spec((_NOBS * _U, _NOBS)),
        ],
        out_specs=[row_spec(_DYN), row_spec(_NOBS)],
        out_shape=[
            jax.ShapeDtypeStruct((B, _DYN), f32),
            jax.ShapeDtypeStruct((B, _NOBS), f32),
        ],
        scratch_shapes=[
            pltpu.VMEM((_DYN, _H1), bf16),
            pltpu.VMEM((_STAT, _H1), bf16),
            pltpu.VMEM((_H1, _H2), bf16),
            pltpu.VMEM((_H2, _HZ), bf16),
            pltpu.VMEM((_HZ, _DYN * _DYN), bf16),
            pltpu.VMEM((_HZ, _DYN * _U), bf16),
            pltpu.VMEM((_HZ, _NOBS * _DYN), bf16),
            pltpu.VMEM((_HZ, _NOBS * _U), bf16),
        ],
        compiler_params=pltpu.CompilerParams(
            dimension_semantics=("arbitrary",)),
    )(z_dyn, z_static, dt, ut,
      W1d, W1s, W1t, b1.reshape(1, -1),
      W2, b2.reshape(1, -1),
      W3, b3.reshape(1, -1),
      Wa, Wb, Wc, Wd,
      baT, bbT, bcT, bdT,
      sa, sb, sc, sd)

    return (z_next, yt)


# speedup vs baseline: 1.4934x; 1.1285x over previous
import jax
import jax.numpy as jnp
from jax.experimental import pallas as pl
from jax.experimental.pallas import tpu as pltpu

_DYN = 96
_STAT = 32
_U = 8
_NOBS = 13
_HZ = 128
_H1 = 200
_H2 = 200
_LANE = 128
_KAB = (_DYN + _U) * _HZ

_TB = 256


def _seg_sum_matrix(n_cols, period, n_out):
    return (jnp.arange(n_cols)[:, None] // period
            == jnp.arange(n_out)[None, :]).astype(jnp.bfloat16)


def _fused_body(zd_ref, zs_ref, dt_ref, ut_ref,
                w1d_ref, w1s_ref, w1t_ref, b1_ref,
                w2_ref, b2_ref, w3_ref, b3_ref,
                wab_ref, wc_ref, wd_ref,
                baT_ref, bbT_ref, bcT_ref, bdT_ref,
                sc_ref, sd_ref,
                z_ref, y_ref,
                w1d_bf, w1s_bf, w2_bf, w3_bf,
                wab_bf, wc_bf, wd_bf):
    bf16 = jnp.bfloat16
    f32 = jnp.float32

    @pl.when(pl.program_id(0) == 0)
    def _cast_weights():
        w1d_bf[...] = w1d_ref[...].astype(bf16)
        w1s_bf[...] = w1s_ref[...].astype(bf16)
        w2_bf[...] = w2_ref[...].astype(bf16)
        w3_bf[...] = w3_ref[...].astype(bf16)
        wab_bf[...] = wab_ref[...].astype(bf16)
        wc_bf[...] = wc_ref[...].astype(bf16)
        wd_bf[...] = wd_ref[...].astype(bf16)

    zd = zd_ref[...]
    dt = dt_ref[...]
    h = (jnp.dot(zd.astype(bf16), w1d_bf[...], preferred_element_type=f32)
         + jnp.dot(zs_ref[...].astype(bf16), w1s_bf[...],
                   preferred_element_type=f32)
         + dt * w1t_ref[...] + b1_ref[...])
    h = jnp.maximum(h, 0.0)
    h = jnp.maximum(
        jnp.dot(h.astype(bf16), w2_bf[...], preferred_element_type=f32)
        + b2_ref[...], 0.0)
    hz = (jnp.dot(h.astype(bf16), w3_bf[...], preferred_element_type=f32)
          + b3_ref[...])
    hzb = hz.astype(bf16)

    utdt = ut_ref[...] * dt
    zbf = zd.astype(bf16)
    ubf = utdt.astype(bf16)

    hparts = [hzb * zbf[:, j:j + 1] for j in range(_DYN)]
    hparts += [hzb * ubf[:, j:j + 1] for j in range(_U)]
    H = jnp.concatenate(hparts, axis=1)
    z_lin = jnp.dot(H, wab_bf[...], preferred_element_type=f32)

    zn = (z_lin
          + jnp.dot(zd, baT_ref[...], preferred_element_type=f32)
          + jnp.dot(utdt, bbT_ref[...], preferred_element_type=f32))

    zn384 = jnp.concatenate([zn] * 4, axis=1).astype(bf16)
    u128 = jnp.concatenate([utdt] * 16, axis=1).astype(bf16)

    ct = jnp.dot(hzb, wc_bf[...],
                 preferred_element_type=f32).astype(bf16)
    p_c = jnp.concatenate(
        [ct[:, k * 384:(k + 1) * 384] * zn384 for k in range(3)]
        + [ct[:, 1152:1248] * zn384[:, :_DYN]], axis=1)
    yt = jnp.dot(p_c, sc_ref[...], preferred_element_type=f32)

    dtv = jnp.dot(hzb, wd_bf[...],
                  preferred_element_type=f32).astype(bf16)
    p_d = dtv * u128[:, :_NOBS * _U]
    yt = (yt + jnp.dot(p_d, sd_ref[...], preferred_element_type=f32)
          + jnp.dot(zn, bcT_ref[...], preferred_element_type=f32)
          + jnp.dot(utdt, bdT_ref[...], preferred_element_type=f32))

    z_ref[...] = zn
    y_ref[...] = yt


@jax.jit
def kernel(z_dyn, z_static, dt, ut, W1, b1, W2, b2, W3, b3,
           Wa, ba, Wb, bb, Wc, bc, Wd, bd):
    B = z_dyn.shape[0]
    f32 = jnp.float32
    bf16 = jnp.bfloat16

    W1d = W1[:_DYN]
    W1s = W1[_DYN:_DYN + _STAT]
    W1t = W1[_DYN + _STAT:]

    Wa_s = Wa.reshape(_HZ, _DYN, _DYN).transpose(2, 0, 1).reshape(
        _DYN * _HZ, _DYN)
    Wb_s = Wb.reshape(_HZ, _DYN, _U).transpose(2, 0, 1).reshape(
        _U * _HZ, _DYN)
    Wab = jnp.concatenate([Wa_s, Wb_s], axis=0)

    baT = ba.reshape(_DYN, _DYN).T
    bbT = bb.reshape(_DYN, _U).T
    bcT = bc.reshape(_NOBS, _DYN).T
    bdT = bd.reshape(_NOBS, _U).T

    sc = _seg_sum_matrix(_NOBS * _DYN, _DYN, _NOBS)
    sd = _seg_sum_matrix(_NOBS * _U, _U, _NOBS)

    grid = (B // _TB,)
    row_spec = lambda n: pl.BlockSpec((_TB, n), lambda i: (i, 0))
    w_spec = lambda shp: pl.BlockSpec(shp, lambda i: (0,) * len(shp))

    z_next, yt = pl.pallas_call(
        _fused_body,
        grid=grid,
        in_specs=[
            row_spec(_DYN),
            row_spec(_STAT),
            row_spec(1),
            row_spec(_U),
            w_spec((_DYN, _H1)), w_spec((_STAT, _H1)), w_spec((1, _H1)),
            w_spec((1, _H1)),
            w_spec((_H1, _H2)), w_spec((1, _H2)),
            w_spec((_H2, _HZ)), w_spec((1, _HZ)),
            w_spec((_KAB, _DYN)),
            w_spec((_HZ, _NOBS * _DYN)),
            w_spec((_HZ, _NOBS * _U)),
            w_spec((_DYN, _DYN)), w_spec((_U, _DYN)),
            w_spec((_DYN, _NOBS)), w_spec((_U, _NOBS)),
            w_spec((_NOBS * _DYN, _NOBS)),
            w_spec((_NOBS * _U, _NOBS)),
        ],
        out_specs=[row_spec(_DYN), row_spec(_NOBS)],
        out_shape=[
            jax.ShapeDtypeStruct((B, _DYN), f32),
            jax.ShapeDtypeStruct((B, _NOBS), f32),
        ],
        scratch_shapes=[
            pltpu.VMEM((_DYN, _H1), bf16),
            pltpu.VMEM((_STAT, _H1), bf16),
            pltpu.VMEM((_H1, _H2), bf16),
            pltpu.VMEM((_H2, _HZ), bf16),
            pltpu.VMEM((_KAB, _DYN), bf16),
            pltpu.VMEM((_HZ, _NOBS * _DYN), bf16),
            pltpu.VMEM((_HZ, _NOBS * _U), bf16),
        ],
        compiler_params=pltpu.CompilerParams(
            dimension_semantics=("arbitrary",)),
    )(z_dyn, z_static, dt, ut,
      W1d, W1s, W1t, b1.reshape(1, -1),
      W2, b2.reshape(1, -1),
      W3, b3.reshape(1, -1),
      Wab, Wc, Wd,
      baT, bbT, bcT, bdT,
      sc, sd)

    return (z_next, yt)
